# baseline (speedup 1.0000x reference)
import jax
import jax.numpy as jnp
from jax.experimental import pallas as pl
from jax.experimental.pallas import tpu as pltpu


def _palette_quant_body(x_ref, m_ref, a_ref, o_ref):
    x = x_ref[0]
    m = m_ref[0]
    a = a_ref[0]

    n = x.shape[1]
    ones = jnp.ones((1, n), dtype=x.dtype)
    xaug = jnp.concatenate([x, ones], axis=0)
    t = jnp.dot(m, xaug, preferred_element_type=jnp.float32)
    e = jnp.exp(t)
    r = jnp.dot(a, e, preferred_element_type=jnp.float32)
    inv = 1.0 / r[3:4, :]
    o_ref[0] = r[0:3, :] * inv


def _deint_matrix(C, L):
    l = jnp.arange(C * L)[:, None]
    n = jnp.arange(C * L)[None, :]
    return jnp.asarray((l % C) * L + l // C == n, jnp.float32)


def kernel(images, palettes, temperature):
    B, H, W, C = images.shape
    K = palettes.shape[1]
    HW = H * W
    N = 2048
    grid = (B, HW // N)
    L = 128
    R = HW // L

    G = _deint_matrix(C, L)
    xg = images.reshape(B * R, C * L) @ G
    xp = (
        xg.reshape(B, R, C, L).transpose(0, 2, 1, 3).reshape(B, C, HW)
    )
    scale = 2.0 / temperature
    bias = -jnp.sum(palettes * palettes, axis=-1) / temperature
    m = jnp.concatenate([palettes * scale, bias[..., None]], axis=-1)
    a = jnp.concatenate(
        [palettes, jnp.ones((B, K, 1), palettes.dtype)], axis=-1
    ).transpose(0, 2, 1)

    out_planar = pl.pallas_call(
        _palette_quant_body,
        grid=grid,
        in_specs=[
            pl.BlockSpec((1, C, N), lambda i, j: (i, 0, j)),
            pl.BlockSpec((1, K, C + 1), lambda i, j: (i, 0, 0)),
            pl.BlockSpec((1, C + 1, K), lambda i, j: (i, 0, 0)),
        ],
        out_specs=pl.BlockSpec((1, C, N), lambda i, j: (i, 0, j)),
        out_shape=jax.ShapeDtypeStruct((B, C, HW), jnp.float32),
    )(xp, m, a)

    og = (
        out_planar.reshape(B, C, R, L).transpose(0, 2, 1, 3).reshape(B * R, C * L)
    )
    return (og @ G.T).reshape(B, H, W, C)


# speedup vs baseline: 1.5499x; 1.5499x over previous
import functools

import jax
import jax.numpy as jnp
from jax.experimental import pallas as pl
from jax.experimental.pallas import tpu as pltpu


def _palette_quant_body(x_ref, g_ref, gt_ref, mbd_ref, a_ref, x_out_ref,
                        *, K, Rm):
    nsub = Rm // 8
    X = x_ref[0]
    G = g_ref[...]
    GT = gt_ref[...]
    M = mbd_ref[0]
    A = a_ref[0]

    XG = jnp.dot(X, G, preferred_element_type=jnp.float32)

    chans = []
    for c in range(3):
        chans.append(jnp.concatenate(
            [XG[8 * r:8 * r + 8, 128 * c:128 * c + 128] for r in range(nsub)],
            axis=1,
        ))
    ones = jnp.ones_like(chans[0])
    xstack = jnp.concatenate(chans + [ones], axis=0)

    T = jnp.dot(M, xstack, preferred_element_type=jnp.float32)
    E = jnp.exp(T)
    rs = jnp.dot(A, E, preferred_element_type=jnp.float32)

    inv = 1.0 / rs[24:32, :]
    o0 = rs[0:8, :] * inv
    o1 = rs[8:16, :] * inv
    o2 = rs[16:24, :] * inv
    out_chunks = []
    for r in range(nsub):
        out_chunks.append(
            jnp.concatenate(
                [o0[:, 128 * r:128 * r + 128],
                 o1[:, 128 * r:128 * r + 128],
                 o2[:, 128 * r:128 * r + 128]],
                axis=1,
            )
        )
    O = jnp.concatenate(out_chunks, axis=0)
    x_out_ref[0] = jnp.dot(O, GT, preferred_element_type=jnp.float32)


def _deint_matrix(C, L):
    l = jnp.arange(C * L)[:, None]
    n = jnp.arange(C * L)[None, :]
    return jnp.asarray((l % C) * L + l // C == n, jnp.float32)


def kernel(images, palettes, temperature):
    B, H, W, C = images.shape
    K = palettes.shape[1]
    HW = H * W
    L = 128
    R = HW // L
    Rm = 64
    grid = (B, R // Rm)

    xr = images.reshape(B, R, C * L)

    G = _deint_matrix(C, L)
    GT = G.T

    scale = 2.0 / temperature
    bias = -jnp.sum(palettes * palettes, axis=-1) / temperature
    m4 = jnp.concatenate([palettes * scale, bias[..., None]], axis=-1)

    eye8 = jnp.eye(8, dtype=palettes.dtype)
    Mbd = (m4[:, :, None, :, None] * eye8[None, None, :, None, :]).reshape(
        B, 8 * K, 32
    )
    q = jnp.concatenate(
        [palettes.transpose(0, 2, 1), jnp.ones((B, 1, K), palettes.dtype)],
        axis=1,
    )
    A = (q[:, :, None, :, None] * eye8[None, None, :, None, :]).reshape(
        B, 32, 8 * K
    )

    out = pl.pallas_call(
        functools.partial(_palette_quant_body, K=K, Rm=Rm),
        grid=grid,
        in_specs=[
            pl.BlockSpec((1, Rm, C * L), lambda i, j: (i, j, 0)),
            pl.BlockSpec((C * L, C * L), lambda i, j: (0, 0)),
            pl.BlockSpec((C * L, C * L), lambda i, j: (0, 0)),
            pl.BlockSpec((1, 8 * K, 32), lambda i, j: (i, 0, 0)),
            pl.BlockSpec((1, 32, 8 * K), lambda i, j: (i, 0, 0)),
        ],
        out_specs=pl.BlockSpec((1, Rm, C * L), lambda i, j: (i, j, 0)),
        out_shape=jax.ShapeDtypeStruct((B, R, C * L), jnp.float32),
    )(xr, G, GT, Mbd, A)

    return out.reshape(B, H, W, C)


# speedup vs baseline: 1.5656x; 1.0101x over previous
import functools

import jax
import jax.numpy as jnp
from jax.experimental import pallas as pl
from jax.experimental.pallas import tpu as pltpu


def _palette_quant_body(x_ref, m_ref, a_ref, o_ref, *, Npx):
    x = x_ref[0].reshape(Npx, 3)
    m = m_ref[0]
    a = a_ref[0]

    xp = x.T
    ones = jnp.ones((1, Npx), dtype=xp.dtype)
    xaug = jnp.concatenate([xp, ones], axis=0)
    t = jnp.dot(m, xaug, preferred_element_type=jnp.float32)
    e = jnp.exp(t)
    r = jnp.dot(a, e, preferred_element_type=jnp.float32)
    inv = 1.0 / r[3:4, :]
    out = r[0:3, :] * inv
    o_ref[0] = out.T.reshape(x_ref.shape[1], x_ref.shape[2], 3)


def kernel(images, palettes, temperature):
    B, H, W, C = images.shape
    K = palettes.shape[1]
    Rh = 8
    Npx = Rh * W
    grid = (B, H // Rh)

    scale = 2.0 / temperature
    bias = -jnp.sum(palettes * palettes, axis=-1) / temperature
    m = jnp.concatenate([palettes * scale, bias[..., None]], axis=-1)
    a = jnp.concatenate(
        [palettes, jnp.ones((B, K, 1), palettes.dtype)], axis=-1
    ).transpose(0, 2, 1)

    out = pl.pallas_call(
        functools.partial(_palette_quant_body, Npx=Npx),
        grid=grid,
        in_specs=[
            pl.BlockSpec((1, Rh, W, C), lambda i, j: (i, j, 0, 0)),
            pl.BlockSpec((1, K, C + 1), lambda i, j: (i, 0, 0)),
            pl.BlockSpec((1, C + 1, K), lambda i, j: (i, 0, 0)),
        ],
        out_specs=pl.BlockSpec((1, Rh, W, C), lambda i, j: (i, j, 0, 0)),
        out_shape=jax.ShapeDtypeStruct((B, H, W, C), jnp.float32),
    )(images, m, a)

    return out
